# baseline (speedup 1.0000x reference)
import functools

import jax
import jax.numpy as jnp
from jax import lax
from jax.experimental import pallas as pl
from jax.experimental.pallas import tpu as pltpu
from jax.experimental.pallas import tpu_sc as plsc

SC_ROWS = 128


def _add_kernel(x_ref, pos_ref, sc_ref, out_ref):
    del sc_ref
    out_ref[...] = x_ref[...] + pos_ref[...][None]


def kernel(x, pos_table):
    B, L, D = x.shape
    NW = 32
    rows_per_w = SC_ROWS // NW
    NBUF = 4
    mesh = plsc.VectorSubcoreMesh(core_axis_name="c", subcore_axis_name="s")

    @functools.partial(
        pl.kernel,
        out_type=jax.ShapeDtypeStruct((B, L, D), jnp.float32),
        mesh=mesh,
        scratch_types=[
            pltpu.VMEM((L, D), jnp.float32),
            pltpu.VMEM((L, D), jnp.float32),
            pltpu.VMEM((L, D), jnp.float32),
            pltpu.VMEM((L, D), jnp.float32),
            pltpu.VMEM((L, D), jnp.float32),
            pltpu.SemaphoreType.DMA,
            pltpu.SemaphoreType.DMA,
            pltpu.SemaphoreType.DMA,
            pltpu.SemaphoreType.DMA,
            pltpu.SemaphoreType.DMA,
            pltpu.SemaphoreType.DMA,
            pltpu.SemaphoreType.DMA,
            pltpu.SemaphoreType.DMA,
        ],
    )
    def sc_add(x_hbm, pos_hbm, out_hbm, pos_v, b0, b1, b2, b3,
               si0, si1, si2, si3, so0, so1, so2, so3):
        wid = lax.axis_index("s") * 2 + lax.axis_index("c")
        base = wid * rows_per_w
        pltpu.sync_copy(pos_hbm, pos_v)

        bufs = (b0, b1, b2, b3)
        isems = (si0, si1, si2, si3)
        osems = (so0, so1, so2, so3)
        h_in = [None] * NBUF
        h_out = [None] * NBUF

        h_in[0] = pltpu.async_copy(x_hbm.at[base + 0], bufs[0], isems[0])
        h_in[1] = pltpu.async_copy(x_hbm.at[base + 1], bufs[1], isems[1])

        for i in range(rows_per_w):
            p = i % NBUF
            nxt = i + 2
            if nxt < rows_per_w:
                q = nxt % NBUF
                if h_out[q] is not None:
                    h_out[q].wait()
                h_in[q] = pltpu.async_copy(x_hbm.at[base + nxt], bufs[q],
                                           isems[q])

            buf = bufs[p]
            h_in[p].wait()

            def add_body(r, c, buf=buf):
                for u in range(8):
                    sl = pl.ds(u * 16, 16)
                    plsc.addupdate(buf.at[r, sl], pos_v[r, sl])
                return c

            lax.fori_loop(0, L, add_body, 0)

            h_out[p] = pltpu.async_copy(buf, out_hbm.at[base + i], osems[p])

        for p in range(NBUF):
            if h_out[p] is not None:
                h_out[p].wait()

    sc_part = sc_add(x, pos_table)

    BLK = 128
    grid = ((B - SC_ROWS) // BLK,)
    return pl.pallas_call(
        _add_kernel,
        grid=grid,
        in_specs=[
            pl.BlockSpec((BLK, L, D), lambda i: (i + 1, 0, 0)),
            pl.BlockSpec((L, D), lambda i: (0, 0)),
            pl.BlockSpec(memory_space=pl.ANY),
        ],
        out_specs=pl.BlockSpec((BLK, L, D), lambda i: (i + 1, 0, 0)),
        out_shape=jax.ShapeDtypeStruct((B, L, D), jnp.float32),
        input_output_aliases={2: 0},
    )(x, pos_table, sc_part)


# speedup vs baseline: 1.4323x; 1.4323x over previous
import jax
from jax.experimental import pallas as pl


def _add_kernel(x_ref, pos_ref, out_ref):
    out_ref[...] = x_ref[...] + pos_ref[...][None]


def kernel(x, pos_table):
    B, L, D = x.shape
    BLK = 128
    grid = (B // BLK,)
    return pl.pallas_call(
        _add_kernel,
        grid=grid,
        in_specs=[
            pl.BlockSpec((BLK, L, D), lambda i: (i, 0, 0)),
            pl.BlockSpec((L, D), lambda i: (0, 0)),
        ],
        out_specs=pl.BlockSpec((BLK, L, D), lambda i: (i, 0, 0)),
        out_shape=jax.ShapeDtypeStruct((B, L, D), x.dtype),
    )(x, pos_table)
